# baseline (speedup 1.0000x reference)
import functools

import jax
import jax.numpy as jnp
from jax import lax
from jax.experimental import pallas as pl
from jax.experimental.pallas import tpu as pltpu
from jax.experimental.pallas import tpu_sc as plsc

_B = 4096
_N = 200
_D = 64
_K = 8
_L = 16
_NC = 2
_NS = 16
_NW = _NC * _NS
_BAGS_PER_W = _B // _NW
_BLK = _L
_NBLK = _BAGS_PER_W // _BLK
_NTILE = _B // _BLK
_TW = _N * _L
_NEG_INF = float("-inf")
_LN2 = 0.6931471805599453
_LNK = 2.0794415416798357

_SORT8 = ((0, 1), (2, 3), (4, 5), (6, 7),
          (0, 2), (1, 3), (4, 6), (5, 7),
          (1, 2), (5, 6), (0, 4), (3, 7),
          (1, 5), (2, 6),
          (1, 4), (3, 6),
          (2, 4), (3, 5),
          (3, 4))
_BITONIC8 = ((0, 4), (1, 5), (2, 6), (3, 7),
             (0, 2), (1, 3), (4, 6), (5, 7),
             (0, 1), (2, 3), (4, 5), (6, 7))


def _log_1_to_8(x):
    bits = lax.bitcast_convert_type(x, jnp.int32)
    e = lax.convert_element_type((bits >> 23) - 127, jnp.float32)
    m = lax.bitcast_convert_type(
        (bits & jnp.int32(0x007FFFFF)) | jnp.int32(0x3F800000), jnp.float32)
    z = (m - 1.0) / (m + 1.0)
    z2 = z * z
    p = 1.0 / 9.0 + z2 * (1.0 / 11.0)
    p = 1.0 / 7.0 + z2 * p
    p = 1.0 / 5.0 + z2 * p
    p = 1.0 / 3.0 + z2 * p
    p = 1.0 + z2 * p
    return e * _LN2 + 2.0 * z * p


@jax.jit
def _sc_call(scores_t, mask_t, reprs_flat):
    mesh = plsc.VectorSubcoreMesh(
        core_axis_name="c", subcore_axis_name="s",
        num_cores=_NC, num_subcores=_NS)

    @functools.partial(
        pl.kernel,
        out_type=(
            jax.ShapeDtypeStruct((_B,), jnp.float32),
            jax.ShapeDtypeStruct((_NTILE * _TW,), jnp.float32),
            jax.ShapeDtypeStruct((_B * _D,), jnp.float32),
        ),
        mesh=mesh,
        compiler_params=pltpu.CompilerParams(use_tc_tiling_on_sc=False,
                                             needs_layout_passes=False),
        scratch_types=[
            pltpu.VMEM((2 * _TW,), jnp.float32),
            pltpu.VMEM((2 * _TW,), jnp.float32),
            pltpu.VMEM((_TW,), jnp.float32),
            pltpu.VMEM((2 * _TW,), jnp.float32),
            pltpu.VMEM((_BLK * _K,), jnp.int32),
            pltpu.VMEM((_BLK * _K, _D), jnp.float32),
            pltpu.VMEM((2 * _BLK * _D,), jnp.float32),
            pltpu.VMEM((_BAGS_PER_W,), jnp.float32),
            pltpu.SemaphoreType.DMA,
            pltpu.SemaphoreType.DMA,
            pltpu.SemaphoreType.DMA,
            pltpu.SemaphoreType.DMA,
        ],
    )
    def sc_kernel(scores_hbm, mask_hbm, reprs_hbm,
                  agg_hbm, w_hbm, rep_hbm,
                  sc_v, mf_v, sm_v, w_v, idx_v, rows_v, rep_v,
                  agg_v, sem_in, sem_g, sem_w, sem_r):
        wid = lax.axis_index("s") * _NC + lax.axis_index("c")
        tile0 = wid * _NBLK
        lanes = lax.iota(jnp.int32, _L)
        zeros_i = jnp.zeros((_L,), jnp.int32)
        zeros_f = jnp.zeros((_L,), jnp.float32)

        def issue_in(blk, par):
            t = tile0 + blk
            pltpu.async_copy(scores_hbm.at[pl.ds(t * _TW, _TW)],
                             sc_v.at[pl.ds(par * _TW, _TW)], sem_in)
            pltpu.async_copy(mask_hbm.at[pl.ds(t * _TW, _TW)],
                             mf_v.at[pl.ds(par * _TW, _TW)], sem_in)

        issue_in(0, 0)

        def cmp_desc(arr, a, b):
            hi = jnp.maximum(arr[a], arr[b])
            lo = jnp.minimum(arr[a], arr[b])
            arr[a], arr[b] = hi, lo

        def block_body(blk, _):
            par = blk % 2
            poff = par * _TW
            tile = tile0 + blk
            row0 = tile * _BLK

            pltpu.make_async_copy(scores_hbm.at[pl.ds(tile * _TW, _TW)],
                                  sc_v.at[pl.ds(poff, _TW)], sem_in).wait()
            pltpu.make_async_copy(mask_hbm.at[pl.ds(tile * _TW, _TW)],
                                  mf_v.at[pl.ds(poff, _TW)], sem_in).wait()

            @pl.when(blk + 1 < _NBLK)
            def _prefetch():
                issue_in(blk + 1, 1 - par)

            def p1_body(g, t):
                t = list(t)
                base = poff + g * (8 * _L)
                d = []
                for ju in range(8):
                    s = sc_v[pl.ds(base + ju * _L, _L)]
                    mf = mf_v[pl.ds(base + ju * _L, _L)]
                    sm = jnp.where(mf > 0.0, s, _NEG_INF)
                    sm_v[pl.ds(g * (8 * _L) + ju * _L, _L)] = sm
                    d.append(sm)
                for a, b in _SORT8:
                    cmp_desc(d, a, b)
                t = [jnp.maximum(t[i], d[7 - i]) for i in range(8)]
                for a, b in _BITONIC8:
                    cmp_desc(t, a, b)
                return tuple(t)

            t = lax.fori_loop(0, _N // 8, p1_body,
                              tuple(zeros_f + _NEG_INF for _ in range(8)),
                              unroll=2)

            big_t = t[0]
            thr = t[7]
            empty = big_t == _NEG_INF
            c_gt = zeros_i
            for i in range(7):
                c_gt = c_gt + jnp.where(t[i] > thr, 1, 0)
            budget = jnp.where(empty, 0, _K - c_gt)

            ssum = zeros_f
            for i in range(8):
                ssum = ssum + jnp.exp(t[i] - big_t)
            agg = big_t + _log_1_to_8(ssum) - _LNK
            agg = jnp.where(empty, 0.0, agg)
            plsc.store_scatter(agg_v, [blk * _BLK + lanes], agg)
            scale_vec = jnp.where(empty, 0.0, 1.0 / _K)

            for c in range(_BLK * _K // _L):
                idx_v[pl.ds(c * _L, _L)] = zeros_i

            @pl.when(blk >= 2)
            def _drain_out():
                pltpu.make_async_copy(w_v.at[pl.ds(poff, _TW)],
                                      w_hbm.at[pl.ds(tile * _TW, _TW)],
                                      sem_w).wait()
                pltpu.make_async_copy(rep_v.at[pl.ds(par * _BLK * _D,
                                                     _BLK * _D)],
                                      rep_hbm.at[pl.ds(row0 * _D, _BLK * _D)],
                                      sem_r).wait()

            gbase = (row0 + lanes) * _N

            def p2_body(j, carry):
                eq_cnt, cnt = carry
                sm = sm_v[pl.ds(j * _L, _L)]
                take_eq = (sm == thr) & (eq_cnt < budget)
                take = (sm > thr) | take_eq
                w_v[pl.ds(poff + j * _L, _L)] = jnp.where(take, 1.0 / _K, 0.0)
                plsc.store_scatter(idx_v, [cnt * _L + lanes],
                                   gbase + j, mask=take)
                eq_cnt = eq_cnt + jnp.where(take_eq, 1, 0)
                cnt = cnt + jnp.where(take, 1, 0)
                return eq_cnt, cnt

            lax.fori_loop(0, _N, p2_body, (zeros_i, zeros_i), unroll=8)

            pltpu.async_copy(w_v.at[pl.ds(poff, _TW)],
                             w_hbm.at[pl.ds(tile * _TW, _TW)], sem_w)

            gcp = pltpu.async_copy(reprs_hbm.at[idx_v], rows_v, sem_g)
            gcp.wait()

            roff = par * _BLK * _D
            for b in range(_BLK):
                scale = scale_vec[b]
                for c in range(_D // _L):
                    acc = rows_v[b, pl.ds(c * _L, _L)]
                    for k in range(1, _K):
                        acc = acc + rows_v[k * _BLK + b, pl.ds(c * _L, _L)]
                    rep_v[pl.ds(roff + b * _D + c * _L, _L)] = acc * scale
            pltpu.async_copy(rep_v.at[pl.ds(roff, _BLK * _D)],
                             rep_hbm.at[pl.ds(row0 * _D, _BLK * _D)], sem_r)
            return _

        lax.fori_loop(0, _NBLK, block_body, 0)

        for blk in (_NBLK - 2, _NBLK - 1):
            par = blk % 2
            tile = tile0 + blk
            pltpu.make_async_copy(w_v.at[pl.ds(par * _TW, _TW)],
                                  w_hbm.at[pl.ds(tile * _TW, _TW)],
                                  sem_w).wait()
            pltpu.make_async_copy(
                rep_v.at[pl.ds(par * _BLK * _D, _BLK * _D)],
                rep_hbm.at[pl.ds(tile * _BLK * _D, _BLK * _D)], sem_r).wait()

        pltpu.sync_copy(agg_v, agg_hbm.at[pl.ds(wid * _BAGS_PER_W,
                                                _BAGS_PER_W)])

    return sc_kernel(scores_t, mask_t, reprs_flat)


def kernel(pair_repr, path_scores, path_reprs, bag_mask, W, b):
    del pair_repr, W, b
    scores_t = path_scores.reshape(_NTILE, _BLK, _N).transpose(0, 2, 1)
    scores_t = scores_t.reshape(_NTILE * _TW)
    mask_t = bag_mask.astype(jnp.float32).reshape(_NTILE, _BLK, _N)
    mask_t = mask_t.transpose(0, 2, 1).reshape(_NTILE * _TW)
    reprs_flat = path_reprs.reshape(_B * _N, _D)
    agg_score, w_t, rep_flat = _sc_call(scores_t, mask_t, reprs_flat)
    weights = w_t.reshape(_NTILE, _N, _BLK).transpose(0, 2, 1)
    weights = weights.reshape(_B, _N)
    agg_repr = rep_flat.reshape(_B, _D)
    return (agg_score, weights, agg_repr)


# speedup vs baseline: 1.0901x; 1.0901x over previous
import functools

import jax
import jax.numpy as jnp
from jax import lax
from jax.experimental import pallas as pl
from jax.experimental.pallas import tpu as pltpu
from jax.experimental.pallas import tpu_sc as plsc

_B = 4096
_N = 200
_D = 64
_K = 8
_L = 16
_NC = 2
_NS = 16
_NW = _NC * _NS
_BAGS_PER_W = _B // _NW
_BLK = _L
_NBLK = _BAGS_PER_W // _BLK
_TW = _N * _L
_ST = 17
_NEG_INF = float("-inf")
_LN2 = 0.6931471805599453
_LNK = 2.0794415416798357

_SORT8 = ((0, 1), (2, 3), (4, 5), (6, 7),
          (0, 2), (1, 3), (4, 6), (5, 7),
          (1, 2), (5, 6), (0, 4), (3, 7),
          (1, 5), (2, 6),
          (1, 4), (3, 6),
          (2, 4), (3, 5),
          (3, 4))
_BITONIC8 = ((0, 4), (1, 5), (2, 6), (3, 7),
             (0, 2), (1, 3), (4, 6), (5, 7),
             (0, 1), (2, 3), (4, 5), (6, 7))

_CHUNKS = tuple(range(0, _N - _L + 1, _L)) + (_N - _L,)


def _log_1_to_8(x):
    bits = lax.bitcast_convert_type(x, jnp.int32)
    e = lax.convert_element_type((bits >> 23) - 127, jnp.float32)
    m = lax.bitcast_convert_type(
        (bits & jnp.int32(0x007FFFFF)) | jnp.int32(0x3F800000), jnp.float32)
    z = (m - 1.0) / (m + 1.0)
    z2 = z * z
    p = 1.0 / 9.0 + z2 * (1.0 / 11.0)
    p = 1.0 / 7.0 + z2 * p
    p = 1.0 / 5.0 + z2 * p
    p = 1.0 / 3.0 + z2 * p
    p = 1.0 + z2 * p
    return e * _LN2 + 2.0 * z * p


@jax.jit
def _sc_call(scores_flat, mask_flat, reprs_flat):
    mesh = plsc.VectorSubcoreMesh(
        core_axis_name="c", subcore_axis_name="s",
        num_cores=_NC, num_subcores=_NS)

    @functools.partial(
        pl.kernel,
        out_type=(
            jax.ShapeDtypeStruct((_B,), jnp.float32),
            jax.ShapeDtypeStruct((_B * _N,), jnp.float32),
            jax.ShapeDtypeStruct((_B * _D,), jnp.float32),
        ),
        mesh=mesh,
        compiler_params=pltpu.CompilerParams(use_tc_tiling_on_sc=False,
                                             needs_layout_passes=False),
        scratch_types=[
            pltpu.VMEM((2 * _TW,), jnp.float32),
            pltpu.VMEM((2 * _TW,), jnp.float32),
            pltpu.VMEM((_N * _ST,), jnp.float32),
            pltpu.VMEM((2 * _TW,), jnp.float32),
            pltpu.VMEM((2 * _BLK * _K,), jnp.int32),
            pltpu.VMEM((_BLK * _K,), jnp.int32),
            pltpu.VMEM((_BLK * _K, _D), jnp.float32),
            pltpu.VMEM((2 * _BLK * _D,), jnp.float32),
            pltpu.VMEM((_BAGS_PER_W,), jnp.float32),
            pltpu.SemaphoreType.DMA,
            pltpu.SemaphoreType.DMA,
            pltpu.SemaphoreType.DMA,
            pltpu.SemaphoreType.DMA,
        ],
    )
    def sc_kernel(scores_hbm, mask_hbm, reprs_hbm,
                  agg_hbm, w_hbm, rep_hbm,
                  sc_v, mf_v, sm_v, w_v, idx_v, gidx_v, rows_v, rep_v,
                  agg_v, sem_in, sem_g, sem_w, sem_r):
        wid = lax.axis_index("s") * _NC + lax.axis_index("c")
        row00 = wid * _BAGS_PER_W
        lanes = lax.iota(jnp.int32, _L)
        l17 = lanes * _ST
        l200 = lanes * _N
        zeros_i = jnp.zeros((_L,), jnp.int32)
        zeros_f = jnp.zeros((_L,), jnp.float32)

        for c in range(2 * _TW // _L):
            w_v[pl.ds(c * _L, _L)] = zeros_f
        for c in range(2 * _BLK * _K // _L):
            idx_v[pl.ds(c * _L, _L)] = zeros_i

        def issue_in(blk, par):
            off = (row00 + blk * _BLK) * _N
            pltpu.async_copy(scores_hbm.at[pl.ds(off, _TW)],
                             sc_v.at[pl.ds(par * _TW, _TW)], sem_in)
            pltpu.async_copy(mask_hbm.at[pl.ds(off, _TW)],
                             mf_v.at[pl.ds(par * _TW, _TW)], sem_in)

        issue_in(0, 0)

        def cmp_desc(arr, a, b):
            hi = jnp.maximum(arr[a], arr[b])
            lo = jnp.minimum(arr[a], arr[b])
            arr[a], arr[b] = hi, lo

        def block_body(blk, _):
            par = blk % 2
            poff = par * _TW
            row0 = row00 + blk * _BLK
            goff = row0 * _N

            pltpu.make_async_copy(scores_hbm.at[pl.ds(goff, _TW)],
                                  sc_v.at[pl.ds(poff, _TW)], sem_in).wait()
            pltpu.make_async_copy(mask_hbm.at[pl.ds(goff, _TW)],
                                  mf_v.at[pl.ds(poff, _TW)], sem_in).wait()

            @pl.when(blk + 1 < _NBLK)
            def _prefetch():
                issue_in(blk + 1, 1 - par)

            def tr_body(b, _c):
                base = poff + b * _N
                for off in _CHUNKS:
                    s = sc_v[pl.ds(base + off, _L)]
                    mf = mf_v[pl.ds(base + off, _L)]
                    sm = jnp.where(mf > 0.0, s, _NEG_INF)
                    plsc.store_scatter(sm_v, [l17 + (off * _ST + b)], sm)
                return _c

            lax.fori_loop(0, _BLK, tr_body, 0, unroll=2)

            def p1_body(g, t):
                t = list(t)
                d = []
                for ju in range(8):
                    d.append(sm_v[pl.ds(g * (8 * _ST) + ju * _ST, _L)])
                for a, b in _SORT8:
                    cmp_desc(d, a, b)
                t = [jnp.maximum(t[i], d[7 - i]) for i in range(8)]
                for a, b in _BITONIC8:
                    cmp_desc(t, a, b)
                return tuple(t)

            t = lax.fori_loop(0, _N // 8, p1_body,
                              tuple(zeros_f + _NEG_INF for _ in range(8)),
                              unroll=2)

            big_t = t[0]
            thr = t[7]
            empty = big_t == _NEG_INF
            c_gt = zeros_i
            for i in range(7):
                c_gt = c_gt + jnp.where(t[i] > thr, 1, 0)
            budget = jnp.where(empty, 0, _K - c_gt)

            ssum = zeros_f
            for i in range(8):
                ssum = ssum + jnp.exp(t[i] - big_t)
            agg = big_t + _log_1_to_8(ssum) - _LNK
            agg = jnp.where(empty, 0.0, agg)
            plsc.store_scatter(agg_v, [blk * _BLK + lanes], agg)
            scale_vec = jnp.where(empty, 0.0, 1.0 / _K)

            @pl.when(blk >= 2)
            def _drain_out():
                pltpu.make_async_copy(w_v.at[pl.ds(poff, _TW)],
                                      w_hbm.at[pl.ds(goff, _TW)],
                                      sem_w).wait()
                pltpu.make_async_copy(rep_v.at[pl.ds(par * _BLK * _D,
                                                     _BLK * _D)],
                                      rep_hbm.at[pl.ds(row0 * _D, _BLK * _D)],
                                      sem_r).wait()
                for k in range(_K):
                    old = idx_v[pl.ds(par * _BLK * _K + k * _L, _L)]
                    plsc.store_scatter(w_v, [(poff + l200) + old], zeros_f)

            def p2_body(j, carry):
                eq_cnt, cnt = carry
                sm = sm_v[pl.ds(j * _ST, _L)]
                take_eq = (sm == thr) & (eq_cnt < budget)
                take = (sm > thr) | take_eq
                plsc.store_scatter(idx_v,
                                   [par * _BLK * _K + cnt * _L + lanes],
                                   zeros_i + j, mask=take)
                eq_cnt = eq_cnt + jnp.where(take_eq, 1, 0)
                cnt = cnt + jnp.where(take, 1, 0)
                return eq_cnt, cnt

            lax.fori_loop(0, _N, p2_body, (zeros_i, zeros_i), unroll=8)

            gbase = goff + l200
            not_empty = jnp.logical_not(empty)
            for k in range(_K):
                loc = idx_v[pl.ds(par * _BLK * _K + k * _L, _L)]
                plsc.store_scatter(w_v, [(poff + l200) + loc],
                                   zeros_f + 1.0 / _K, mask=not_empty)
                gidx_v[pl.ds(k * _L, _L)] = gbase + loc

            pltpu.async_copy(w_v.at[pl.ds(poff, _TW)],
                             w_hbm.at[pl.ds(goff, _TW)], sem_w)

            pltpu.async_copy(reprs_hbm.at[gidx_v], rows_v, sem_g).wait()

            roff = par * _BLK * _D
            for b in range(_BLK):
                scale = scale_vec[b]
                for c in range(_D // _L):
                    acc = rows_v[b, pl.ds(c * _L, _L)]
                    for k in range(1, _K):
                        acc = acc + rows_v[k * _BLK + b, pl.ds(c * _L, _L)]
                    rep_v[pl.ds(roff + b * _D + c * _L, _L)] = acc * scale
            pltpu.async_copy(rep_v.at[pl.ds(roff, _BLK * _D)],
                             rep_hbm.at[pl.ds(row0 * _D, _BLK * _D)], sem_r)
            return _

        lax.fori_loop(0, _NBLK, block_body, 0)

        for blk in (_NBLK - 2, _NBLK - 1):
            par = blk % 2
            row0 = row00 + blk * _BLK
            pltpu.make_async_copy(w_v.at[pl.ds(par * _TW, _TW)],
                                  w_hbm.at[pl.ds(row0 * _N, _TW)],
                                  sem_w).wait()
            pltpu.make_async_copy(
                rep_v.at[pl.ds(par * _BLK * _D, _BLK * _D)],
                rep_hbm.at[pl.ds(row0 * _D, _BLK * _D)], sem_r).wait()

        pltpu.sync_copy(agg_v, agg_hbm.at[pl.ds(row00, _BAGS_PER_W)])

    return sc_kernel(scores_flat, mask_flat, reprs_flat)


def kernel(pair_repr, path_scores, path_reprs, bag_mask, W, b):
    del pair_repr, W, b
    scores_flat = path_scores.reshape(_B * _N)
    mask_flat = bag_mask.astype(jnp.float32).reshape(_B * _N)
    reprs_flat = path_reprs.reshape(_B * _N, _D)
    agg_score, w_flat, rep_flat = _sc_call(scores_flat, mask_flat, reprs_flat)
    return (agg_score, w_flat.reshape(_B, _N), rep_flat.reshape(_B, _D))


# speedup vs baseline: 3.0640x; 2.8107x over previous
import functools

import jax
import jax.numpy as jnp
from jax import lax
from jax.experimental import pallas as pl
from jax.experimental.pallas import tpu as pltpu
from jax.experimental.pallas import tpu_sc as plsc

_B = 4096
_N = 200
_D = 64
_K = 8
_L = 16
_NC = 2
_NS = 16
_NW = _NC * _NS
_BAGS_PER_W = _B // _NW
_BLK = _L
_NBLK = _BAGS_PER_W // _BLK
_TW = _N * _L
_ST = 17
_NEG_INF = float("-inf")
_LN2 = 0.6931471805599453
_LNK = 2.0794415416798357

_BL = 128
_NB = 40

_SORT8 = ((0, 1), (2, 3), (4, 5), (6, 7),
          (0, 2), (1, 3), (4, 6), (5, 7),
          (1, 2), (5, 6), (0, 4), (3, 7),
          (1, 5), (2, 6),
          (1, 4), (3, 6),
          (2, 4), (3, 5),
          (3, 4))
_BITONIC8 = ((0, 4), (1, 5), (2, 6), (3, 7),
             (0, 2), (1, 3), (4, 6), (5, 7),
             (0, 1), (2, 3), (4, 5), (6, 7))

_CHUNKS = tuple(range(0, _N - _L + 1, _L)) + (_N - _L,)


def _log_1_to_8(x):
    bits = lax.bitcast_convert_type(x, jnp.int32)
    e = lax.convert_element_type((bits >> 23) - 127, jnp.float32)
    m = lax.bitcast_convert_type(
        (bits & jnp.int32(0x007FFFFF)) | jnp.int32(0x3F800000), jnp.float32)
    z = (m - 1.0) / (m + 1.0)
    z2 = z * z
    p = 1.0 / 9.0 + z2 * (1.0 / 11.0)
    p = 1.0 / 7.0 + z2 * p
    p = 1.0 / 5.0 + z2 * p
    p = 1.0 / 3.0 + z2 * p
    p = 1.0 + z2 * p
    return e * _LN2 + 2.0 * z * p


def _sc_topk(scores_flat, mask_flat):
    mesh = plsc.VectorSubcoreMesh(
        core_axis_name="c", subcore_axis_name="s",
        num_cores=_NC, num_subcores=_NS)

    @functools.partial(
        pl.kernel,
        out_type=(
            jax.ShapeDtypeStruct((_B,), jnp.float32),
            jax.ShapeDtypeStruct((_N, _B), jnp.float32),
        ),
        mesh=mesh,
        compiler_params=pltpu.CompilerParams(use_tc_tiling_on_sc=False,
                                             needs_layout_passes=False),
        scratch_types=[
            pltpu.VMEM((2 * _TW,), jnp.float32),
            pltpu.VMEM((2 * _TW,), jnp.float32),
            pltpu.VMEM((_N * _ST,), jnp.float32),
            pltpu.VMEM((2, _N, _L), jnp.float32),
            pltpu.VMEM((_BAGS_PER_W,), jnp.float32),
            pltpu.SemaphoreType.DMA,
            pltpu.SemaphoreType.DMA,
        ],
    )
    def sc_kernel(scores_hbm, mask_hbm, agg_hbm, wt_hbm,
                  sc_v, mf_v, sm_v, wt_v, agg_v, sem_in, sem_w):
        wid = lax.axis_index("s") * _NC + lax.axis_index("c")
        row00 = wid * _BAGS_PER_W
        lanes = lax.iota(jnp.int32, _L)
        l17 = lanes * _ST
        zeros_i = jnp.zeros((_L,), jnp.int32)
        zeros_f = jnp.zeros((_L,), jnp.float32)

        def issue_in(blk, par):
            off = (row00 + blk * _BLK) * _N
            pltpu.async_copy(scores_hbm.at[pl.ds(off, _TW)],
                             sc_v.at[pl.ds(par * _TW, _TW)], sem_in)
            pltpu.async_copy(mask_hbm.at[pl.ds(off, _TW)],
                             mf_v.at[pl.ds(par * _TW, _TW)], sem_in)

        issue_in(0, 0)

        def cmp_desc(arr, a, b):
            hi = jnp.maximum(arr[a], arr[b])
            lo = jnp.minimum(arr[a], arr[b])
            arr[a], arr[b] = hi, lo

        def block_body(blk, _):
            par = blk % 2
            poff = par * _TW
            row0 = row00 + blk * _BLK
            goff = row0 * _N

            pltpu.make_async_copy(scores_hbm.at[pl.ds(goff, _TW)],
                                  sc_v.at[pl.ds(poff, _TW)], sem_in).wait()
            pltpu.make_async_copy(mask_hbm.at[pl.ds(goff, _TW)],
                                  mf_v.at[pl.ds(poff, _TW)], sem_in).wait()

            @pl.when(blk + 1 < _NBLK)
            def _prefetch():
                issue_in(blk + 1, 1 - par)

            def tr_body(b, _c):
                base = poff + b * _N
                for off in _CHUNKS:
                    s = sc_v[pl.ds(base + off, _L)]
                    mf = mf_v[pl.ds(base + off, _L)]
                    sm = jnp.where(mf > 0.0, s, _NEG_INF)
                    plsc.store_scatter(sm_v, [l17 + (off * _ST + b)], sm)
                return _c

            lax.fori_loop(0, _BLK, tr_body, 0, unroll=2)

            def p1_body(g, t):
                t = list(t)
                d = []
                for ju in range(8):
                    d.append(sm_v[pl.ds(g * (8 * _ST) + ju * _ST, _L)])
                for a, b in _SORT8:
                    cmp_desc(d, a, b)
                t = [jnp.maximum(t[i], d[7 - i]) for i in range(8)]
                for a, b in _BITONIC8:
                    cmp_desc(t, a, b)
                return tuple(t)

            t = lax.fori_loop(0, _N // 8, p1_body,
                              tuple(zeros_f + _NEG_INF for _ in range(8)),
                              unroll=2)

            big_t = t[0]
            thr = t[7]
            empty = big_t == _NEG_INF
            c_gt = zeros_i
            for i in range(7):
                c_gt = c_gt + jnp.where(t[i] > thr, 1, 0)
            budget = jnp.where(empty, 0, _K - c_gt)

            ssum = zeros_f
            for i in range(8):
                ssum = ssum + jnp.exp(t[i] - big_t)
            agg = big_t + _log_1_to_8(ssum) - _LNK
            agg = jnp.where(empty, 0.0, agg)
            plsc.store_scatter(agg_v, [blk * _BLK + lanes], agg)

            @pl.when(blk >= 2)
            def _drain_out():
                pltpu.make_async_copy(
                    wt_v.at[par], wt_hbm.at[:, pl.ds(row0, _BLK)],
                    sem_w).wait()

            def p2_body(j, eq_cnt):
                sm = sm_v[pl.ds(j * _ST, _L)]
                take_eq = (sm == thr) & (eq_cnt < budget)
                take = (sm > thr) | take_eq
                wt_v[par, j, :] = jnp.where(take, 1.0 / _K, 0.0)
                return eq_cnt + jnp.where(take_eq, 1, 0)

            lax.fori_loop(0, _N, p2_body, zeros_i, unroll=8)

            pltpu.async_copy(wt_v.at[par], wt_hbm.at[:, pl.ds(row0, _BLK)],
                             sem_w)
            return _

        lax.fori_loop(0, _NBLK, block_body, 0)

        for blk in (_NBLK - 2, _NBLK - 1):
            par = blk % 2
            row0 = row00 + blk * _BLK
            pltpu.make_async_copy(wt_v.at[par],
                                  wt_hbm.at[:, pl.ds(row0, _BLK)],
                                  sem_w).wait()

        pltpu.sync_copy(agg_v, agg_hbm.at[pl.ds(row00, _BAGS_PER_W)])

    return sc_kernel(scores_flat, mask_flat)


def _tc_pool_kernel(wt_ref, r_ref, out_ref):
    nb = pl.program_id(1)

    @pl.when(nb == 0)
    def _init():
        out_ref[...] = jnp.zeros_like(out_ref)

    acc = out_ref[...]
    for n in range(_NB):
        acc = acc + wt_ref[n, :][None, :] * r_ref[n]
    out_ref[...] = acc


def _tc_pool(wt, reprs_t):
    grid = (_B // _BL, _N // _NB)
    return pl.pallas_call(
        _tc_pool_kernel,
        grid=grid,
        in_specs=[
            pl.BlockSpec((_NB, _BL), lambda b, n: (n, b)),
            pl.BlockSpec((_NB, _D, _BL), lambda b, n: (n, 0, b)),
        ],
        out_specs=pl.BlockSpec((_D, _BL), lambda b, n: (0, b)),
        out_shape=jax.ShapeDtypeStruct((_D, _B), jnp.float32),
        compiler_params=pltpu.CompilerParams(
            dimension_semantics=("arbitrary", "arbitrary")),
    )(wt, reprs_t)


@jax.jit
def _run(path_scores, bag_mask, path_reprs):
    scores_flat = path_scores.reshape(_B * _N)
    mask_flat = bag_mask.astype(jnp.float32).reshape(_B * _N)
    reprs_t = jnp.transpose(path_reprs, (1, 2, 0))
    agg_score, wt = _sc_topk(scores_flat, mask_flat)
    rep_t = _tc_pool(wt, reprs_t)
    return agg_score, wt.T, rep_t.T


def kernel(pair_repr, path_scores, path_reprs, bag_mask, W, b):
    del pair_repr, W, b
    return tuple(_run(path_scores, bag_mask, path_reprs))


# speedup vs baseline: 4.4846x; 1.4636x over previous
import functools

import jax
import jax.numpy as jnp
from jax import lax
from jax.experimental import pallas as pl
from jax.experimental.pallas import tpu as pltpu
from jax.experimental.pallas import tpu_sc as plsc

_B = 4096
_N = 200
_D = 64
_K = 8
_L = 16
_NC = 2
_NS = 16
_NW = _NC * _NS
_BAGS_PER_W = _B // _NW
_BLK = _L
_NBLK = _BAGS_PER_W // _BLK
_TW = _N * _L
_ST = 17
_NEG_INF = float("-inf")
_LN2 = 0.6931471805599453
_LNK = 2.0794415416798357

_BL = 128
_NB = 40

_SORT8 = ((0, 1), (2, 3), (4, 5), (6, 7),
          (0, 2), (1, 3), (4, 6), (5, 7),
          (1, 2), (5, 6), (0, 4), (3, 7),
          (1, 5), (2, 6),
          (1, 4), (3, 6),
          (2, 4), (3, 5),
          (3, 4))
_BITONIC8 = ((0, 4), (1, 5), (2, 6), (3, 7),
             (0, 2), (1, 3), (4, 6), (5, 7),
             (0, 1), (2, 3), (4, 5), (6, 7))

_CHUNKS = tuple(range(0, _N - _L + 1, _L)) + (_N - _L,)


def _log_1_to_8(x):
    bits = lax.bitcast_convert_type(x, jnp.int32)
    e = lax.convert_element_type((bits >> 23) - 127, jnp.float32)
    m = lax.bitcast_convert_type(
        (bits & jnp.int32(0x007FFFFF)) | jnp.int32(0x3F800000), jnp.float32)
    z = (m - 1.0) / (m + 1.0)
    z2 = z * z
    p = 1.0 / 9.0 + z2 * (1.0 / 11.0)
    p = 1.0 / 7.0 + z2 * p
    p = 1.0 / 5.0 + z2 * p
    p = 1.0 / 3.0 + z2 * p
    p = 1.0 + z2 * p
    return e * _LN2 + 2.0 * z * p


def _sc_topk(scores_flat, mask_flat):
    mesh = plsc.VectorSubcoreMesh(
        core_axis_name="c", subcore_axis_name="s",
        num_cores=_NC, num_subcores=_NS)

    @functools.partial(
        pl.kernel,
        out_type=(
            jax.ShapeDtypeStruct((_B,), jnp.float32),
            jax.ShapeDtypeStruct((_N, _B), jnp.float32),
        ),
        mesh=mesh,
        compiler_params=pltpu.CompilerParams(use_tc_tiling_on_sc=False,
                                             needs_layout_passes=False),
        scratch_types=[
            pltpu.VMEM((2 * _TW,), jnp.float32),
            pltpu.VMEM((2 * _TW,), jnp.float32),
            pltpu.VMEM((_N * _ST,), jnp.float32),
            pltpu.VMEM((2, _N, _L), jnp.float32),
            pltpu.VMEM((_BAGS_PER_W,), jnp.float32),
            pltpu.SemaphoreType.DMA,
            pltpu.SemaphoreType.DMA,
        ],
    )
    def sc_kernel(scores_hbm, mask_hbm, agg_hbm, wt_hbm,
                  sc_v, mf_v, sm_v, wt_v, agg_v, sem_in, sem_w):
        wid = lax.axis_index("s") * _NC + lax.axis_index("c")
        row00 = wid * _BAGS_PER_W
        lanes = lax.iota(jnp.int32, _L)
        l17 = lanes * _ST
        zeros_i = jnp.zeros((_L,), jnp.int32)
        zeros_f = jnp.zeros((_L,), jnp.float32)

        def issue_in(blk, par):
            off = (row00 + blk * _BLK) * _N
            pltpu.async_copy(scores_hbm.at[pl.ds(off, _TW)],
                             sc_v.at[pl.ds(par * _TW, _TW)], sem_in)
            pltpu.async_copy(mask_hbm.at[pl.ds(off, _TW)],
                             mf_v.at[pl.ds(par * _TW, _TW)], sem_in)

        issue_in(0, 0)

        def cmp_desc(arr, a, b):
            hi = jnp.maximum(arr[a], arr[b])
            lo = jnp.minimum(arr[a], arr[b])
            arr[a], arr[b] = hi, lo

        def block_body(blk, _):
            par = blk % 2
            poff = par * _TW
            row0 = row00 + blk * _BLK
            goff = row0 * _N

            pltpu.make_async_copy(scores_hbm.at[pl.ds(goff, _TW)],
                                  sc_v.at[pl.ds(poff, _TW)], sem_in).wait()
            pltpu.make_async_copy(mask_hbm.at[pl.ds(goff, _TW)],
                                  mf_v.at[pl.ds(poff, _TW)], sem_in).wait()

            @pl.when(blk + 1 < _NBLK)
            def _prefetch():
                issue_in(blk + 1, 1 - par)

            def tr_body(b, _c):
                base = poff + b * _N
                for off in _CHUNKS:
                    s = sc_v[pl.ds(base + off, _L)]
                    mf = mf_v[pl.ds(base + off, _L)]
                    sm = jnp.where(mf > 0.0, s, _NEG_INF)
                    plsc.store_scatter(sm_v, [l17 + (off * _ST + b)], sm)
                return _c

            lax.fori_loop(0, _BLK, tr_body, 0, unroll=2)

            def p1_body(g, t):
                t = list(t)
                d = []
                for ju in range(8):
                    d.append(sm_v[pl.ds(g * (8 * _ST) + ju * _ST, _L)])
                for a, b in _SORT8:
                    cmp_desc(d, a, b)
                t = [jnp.maximum(t[i], d[7 - i]) for i in range(8)]
                for a, b in _BITONIC8:
                    cmp_desc(t, a, b)
                return tuple(t)

            t = lax.fori_loop(0, _N // 8, p1_body,
                              tuple(zeros_f + _NEG_INF for _ in range(8)),
                              unroll=2)

            big_t = t[0]
            thr = t[7]
            empty = big_t == _NEG_INF
            c_gt = zeros_i
            for i in range(7):
                c_gt = c_gt + jnp.where(t[i] > thr, 1, 0)
            budget = jnp.where(empty, 0, _K - c_gt)

            ssum = zeros_f
            for i in range(8):
                ssum = ssum + jnp.exp(t[i] - big_t)
            agg = big_t + _log_1_to_8(ssum) - _LNK
            agg = jnp.where(empty, 0.0, agg)
            plsc.store_scatter(agg_v, [blk * _BLK + lanes], agg)

            @pl.when(blk >= 2)
            def _drain_out():
                pltpu.make_async_copy(
                    wt_v.at[par], wt_hbm.at[:, pl.ds(row0, _BLK)],
                    sem_w).wait()

            def p2_body(j, eq_cnt):
                sm = sm_v[pl.ds(j * _ST, _L)]
                take_eq = (sm == thr) & (eq_cnt < budget)
                take = (sm > thr) | take_eq
                wt_v[par, j, :] = jnp.where(take, 1.0 / _K, 0.0)
                return eq_cnt + jnp.where(take_eq, 1, 0)

            lax.fori_loop(0, _N, p2_body, zeros_i, unroll=8)

            pltpu.async_copy(wt_v.at[par], wt_hbm.at[:, pl.ds(row0, _BLK)],
                             sem_w)
            return _

        lax.fori_loop(0, _NBLK, block_body, 0)

        for blk in (_NBLK - 2, _NBLK - 1):
            par = blk % 2
            row0 = row00 + blk * _BLK
            pltpu.make_async_copy(wt_v.at[par],
                                  wt_hbm.at[:, pl.ds(row0, _BLK)],
                                  sem_w).wait()

        pltpu.sync_copy(agg_v, agg_hbm.at[pl.ds(row00, _BAGS_PER_W)])

    return sc_kernel(scores_flat, mask_flat)


def _tc_pool_kernel(wt_ref, r_ref, out_ref):
    acc = wt_ref[0, :][None, :] * r_ref[0]
    for n in range(1, _N):
        acc = acc + wt_ref[n, :][None, :] * r_ref[n]
    out_ref[...] = acc


def _tc_pool(wt, reprs_t):
    return pl.pallas_call(
        _tc_pool_kernel,
        grid=(_B // _BL,),
        in_specs=[
            pl.BlockSpec((_N, _BL), lambda b: (0, b)),
            pl.BlockSpec((_N, _D, _BL), lambda b: (0, 0, b)),
        ],
        out_specs=pl.BlockSpec((_D, _BL), lambda b: (0, b)),
        out_shape=jax.ShapeDtypeStruct((_D, _B), jnp.float32),
        compiler_params=pltpu.CompilerParams(
            dimension_semantics=("arbitrary",)),
    )(wt, reprs_t)


@jax.jit
def _run(path_scores, bag_mask, path_reprs):
    scores_flat = path_scores.reshape(_B * _N)
    mask_flat = bag_mask.astype(jnp.float32).reshape(_B * _N)
    reprs_t = jnp.transpose(path_reprs, (1, 2, 0))
    agg_score, wt = _sc_topk(scores_flat, mask_flat)
    rep_t = _tc_pool(wt, reprs_t)
    return agg_score, wt.T, rep_t.T


def kernel(pair_repr, path_scores, path_reprs, bag_mask, W, b):
    del pair_repr, W, b
    return tuple(_run(path_scores, bag_mask, path_reprs))


# speedup vs baseline: 4.4903x; 1.0013x over previous
import functools

import jax
import jax.numpy as jnp
from jax import lax
from jax.experimental import pallas as pl
from jax.experimental.pallas import tpu as pltpu
from jax.experimental.pallas import tpu_sc as plsc

_B = 4096
_N = 200
_D = 64
_K = 8
_L = 16
_NC = 2
_NS = 16
_NW = _NC * _NS
_BAGS_PER_W = _B // _NW
_BLK = _L
_NBLK = _BAGS_PER_W // _BLK
_TW = _N * _L
_ST = 17
_NEG_INF = float("-inf")
_LN2 = 0.6931471805599453
_LNK = 2.0794415416798357

_BL = 256
_NB = 40

_SORT8 = ((0, 1), (2, 3), (4, 5), (6, 7),
          (0, 2), (1, 3), (4, 6), (5, 7),
          (1, 2), (5, 6), (0, 4), (3, 7),
          (1, 5), (2, 6),
          (1, 4), (3, 6),
          (2, 4), (3, 5),
          (3, 4))
_BITONIC8 = ((0, 4), (1, 5), (2, 6), (3, 7),
             (0, 2), (1, 3), (4, 6), (5, 7),
             (0, 1), (2, 3), (4, 5), (6, 7))

_CHUNKS = tuple(range(0, _N - _L + 1, _L)) + (_N - _L,)


def _log_1_to_8(x):
    bits = lax.bitcast_convert_type(x, jnp.int32)
    e = lax.convert_element_type((bits >> 23) - 127, jnp.float32)
    m = lax.bitcast_convert_type(
        (bits & jnp.int32(0x007FFFFF)) | jnp.int32(0x3F800000), jnp.float32)
    z = (m - 1.0) / (m + 1.0)
    z2 = z * z
    p = 1.0 / 9.0 + z2 * (1.0 / 11.0)
    p = 1.0 / 7.0 + z2 * p
    p = 1.0 / 5.0 + z2 * p
    p = 1.0 / 3.0 + z2 * p
    p = 1.0 + z2 * p
    return e * _LN2 + 2.0 * z * p


def _sc_topk(scores_flat, mask_flat):
    mesh = plsc.VectorSubcoreMesh(
        core_axis_name="c", subcore_axis_name="s",
        num_cores=_NC, num_subcores=_NS)

    @functools.partial(
        pl.kernel,
        out_type=(
            jax.ShapeDtypeStruct((_B,), jnp.float32),
            jax.ShapeDtypeStruct((_N, _B), jnp.float32),
        ),
        mesh=mesh,
        compiler_params=pltpu.CompilerParams(use_tc_tiling_on_sc=False,
                                             needs_layout_passes=False),
        scratch_types=[
            pltpu.VMEM((2 * _TW,), jnp.float32),
            pltpu.VMEM((2 * _TW,), jnp.float32),
            pltpu.VMEM((_N * _ST,), jnp.float32),
            pltpu.VMEM((2, _N, _L), jnp.float32),
            pltpu.VMEM((_BAGS_PER_W,), jnp.float32),
            pltpu.SemaphoreType.DMA,
            pltpu.SemaphoreType.DMA,
        ],
    )
    def sc_kernel(scores_hbm, mask_hbm, agg_hbm, wt_hbm,
                  sc_v, mf_v, sm_v, wt_v, agg_v, sem_in, sem_w):
        wid = lax.axis_index("s") * _NC + lax.axis_index("c")
        row00 = wid * _BAGS_PER_W
        lanes = lax.iota(jnp.int32, _L)
        l17 = lanes * _ST
        zeros_i = jnp.zeros((_L,), jnp.int32)
        zeros_f = jnp.zeros((_L,), jnp.float32)

        def issue_in(blk, par):
            off = (row00 + blk * _BLK) * _N
            pltpu.async_copy(scores_hbm.at[pl.ds(off, _TW)],
                             sc_v.at[pl.ds(par * _TW, _TW)], sem_in)
            pltpu.async_copy(mask_hbm.at[pl.ds(off, _TW)],
                             mf_v.at[pl.ds(par * _TW, _TW)], sem_in)

        issue_in(0, 0)

        def cmp_desc(arr, a, b):
            hi = jnp.maximum(arr[a], arr[b])
            lo = jnp.minimum(arr[a], arr[b])
            arr[a], arr[b] = hi, lo

        def block_body(blk, _):
            par = blk % 2
            poff = par * _TW
            row0 = row00 + blk * _BLK
            goff = row0 * _N

            pltpu.make_async_copy(scores_hbm.at[pl.ds(goff, _TW)],
                                  sc_v.at[pl.ds(poff, _TW)], sem_in).wait()
            pltpu.make_async_copy(mask_hbm.at[pl.ds(goff, _TW)],
                                  mf_v.at[pl.ds(poff, _TW)], sem_in).wait()

            @pl.when(blk + 1 < _NBLK)
            def _prefetch():
                issue_in(blk + 1, 1 - par)

            def tr_body(b, _c):
                base = poff + b * _N
                for off in _CHUNKS:
                    s = sc_v[pl.ds(base + off, _L)]
                    mf = mf_v[pl.ds(base + off, _L)]
                    sm = jnp.where(mf > 0.0, s, _NEG_INF)
                    plsc.store_scatter(sm_v, [l17 + (off * _ST + b)], sm)
                return _c

            lax.fori_loop(0, _BLK, tr_body, 0, unroll=2)

            def p1_body(g, t):
                t = list(t)
                d = []
                for ju in range(8):
                    d.append(sm_v[pl.ds(g * (8 * _ST) + ju * _ST, _L)])
                for a, b in _SORT8:
                    cmp_desc(d, a, b)
                t = [jnp.maximum(t[i], d[7 - i]) for i in range(8)]
                for a, b in _BITONIC8:
                    cmp_desc(t, a, b)
                return tuple(t)

            t = lax.fori_loop(0, _N // 8, p1_body,
                              tuple(zeros_f + _NEG_INF for _ in range(8)),
                              unroll=2)

            big_t = t[0]
            thr = t[7]
            empty = big_t == _NEG_INF
            c_gt = zeros_i
            for i in range(7):
                c_gt = c_gt + jnp.where(t[i] > thr, 1, 0)
            budget = jnp.where(empty, 0, _K - c_gt)

            ssum = zeros_f
            for i in range(8):
                ssum = ssum + jnp.exp(t[i] - big_t)
            agg = big_t + _log_1_to_8(ssum) - _LNK
            agg = jnp.where(empty, 0.0, agg)
            plsc.store_scatter(agg_v, [blk * _BLK + lanes], agg)

            @pl.when(blk >= 2)
            def _drain_out():
                pltpu.make_async_copy(
                    wt_v.at[par], wt_hbm.at[:, pl.ds(row0, _BLK)],
                    sem_w).wait()

            def p2_body(j, eq_cnt):
                sm = sm_v[pl.ds(j * _ST, _L)]
                take_eq = (sm == thr) & (eq_cnt < budget)
                take = (sm > thr) | take_eq
                wt_v[par, j, :] = jnp.where(take, 1.0 / _K, 0.0)
                return eq_cnt + jnp.where(take_eq, 1, 0)

            lax.fori_loop(0, _N, p2_body, zeros_i, unroll=8)

            pltpu.async_copy(wt_v.at[par], wt_hbm.at[:, pl.ds(row0, _BLK)],
                             sem_w)
            return _

        lax.fori_loop(0, _NBLK, block_body, 0)

        for blk in (_NBLK - 2, _NBLK - 1):
            par = blk % 2
            row0 = row00 + blk * _BLK
            pltpu.make_async_copy(wt_v.at[par],
                                  wt_hbm.at[:, pl.ds(row0, _BLK)],
                                  sem_w).wait()

        pltpu.sync_copy(agg_v, agg_hbm.at[pl.ds(row00, _BAGS_PER_W)])

    return sc_kernel(scores_flat, mask_flat)


def _tc_pool_kernel(wt_ref, r_ref, out_ref):
    acc = wt_ref[0, :][None, :] * r_ref[0]
    for n in range(1, _N):
        acc = acc + wt_ref[n, :][None, :] * r_ref[n]
    out_ref[...] = acc


def _tc_pool(wt, reprs_t):
    return pl.pallas_call(
        _tc_pool_kernel,
        grid=(_B // _BL,),
        in_specs=[
            pl.BlockSpec((_N, _BL), lambda b: (0, b)),
            pl.BlockSpec((_N, _D, _BL), lambda b: (0, 0, b)),
        ],
        out_specs=pl.BlockSpec((_D, _BL), lambda b: (0, b)),
        out_shape=jax.ShapeDtypeStruct((_D, _B), jnp.float32),
        compiler_params=pltpu.CompilerParams(
            dimension_semantics=("arbitrary",)),
    )(wt, reprs_t)


@jax.jit
def _run(path_scores, bag_mask, path_reprs):
    scores_flat = path_scores.reshape(_B * _N)
    mask_flat = bag_mask.astype(jnp.float32).reshape(_B * _N)
    reprs_t = jnp.transpose(path_reprs, (1, 2, 0))
    agg_score, wt = _sc_topk(scores_flat, mask_flat)
    rep_t = _tc_pool(wt, reprs_t)
    return agg_score, wt.T, rep_t.T


def kernel(pair_repr, path_scores, path_reprs, bag_mask, W, b):
    del pair_repr, W, b
    return tuple(_run(path_scores, bag_mask, path_reprs))
